# baseline (speedup 1.0000x reference)
import functools

import jax
import jax.numpy as jnp
from jax import lax
from jax.experimental import pallas as pl
from jax.experimental.pallas import tpu as pltpu
from jax.experimental.pallas import tpu_sc as plsc

N, E, D, H, O = 10000, 320000, 128, 64, 2
H2 = 2 * H
NC, NS = 2, 16
NW = NC * NS
EPW = E // NW
CHUNK = 125
NCHUNK = EPW // CHUNK
DEG_T = 10
DEG_R = N // DEG_T


def _project_body(x_ref, wl_ref, wr_ref, bl_ref, yr_ref):
    xb = x_ref[...]
    y = jnp.dot(xb, wl_ref[...], preferred_element_type=jnp.float32)
    r = (
        jnp.dot(xb, wr_ref[...], preferred_element_type=jnp.float32)
        + bl_ref[...][None, :]
    )
    yr_ref[...] = jnp.concatenate([y, r], axis=1)


def _project(x, W_l, W_r, b_l):
    BN = 2000
    return pl.pallas_call(
        _project_body,
        grid=(N // BN,),
        in_specs=[
            pl.BlockSpec((BN, D), lambda i: (i, 0)),
            pl.BlockSpec((D, H), lambda i: (0, 0)),
            pl.BlockSpec((D, H), lambda i: (0, 0)),
            pl.BlockSpec((H,), lambda i: (0,)),
        ],
        out_specs=pl.BlockSpec((BN, H2), lambda i: (i, 0)),
        out_shape=jax.ShapeDtypeStruct((N, H2), jnp.float32),
    )(x, W_l, W_r, b_l)


NBUF = 2
LA = NBUF // 2


def _sc_body(y_hbm, edges_hbm, z2_hbm, z1_hbm,
             acc_out, deg0_out, deg1_out,
             src_v, dst_v, rows_v, ones_v, dbuf_v, acc_sh, deg_sh, y_sh,
             *sems):
    gsems = sems[:NBUF]
    ssems = sems[NBUF:2 * NBUF]
    dsems = sems[2 * NBUF:]
    cid = lax.axis_index("c")
    sid = lax.axis_index("s")
    wid = sid * NC + cid

    @pl.when(sid < DEG_T)
    def _():
        pltpu.sync_copy(z2_hbm, acc_sh.at[pl.ds(sid * DEG_R, DEG_R)])
        pltpu.sync_copy(y_hbm.at[pl.ds(sid * DEG_R, DEG_R), pl.ds(0, H)],
                        y_sh.at[pl.ds(sid * DEG_R, DEG_R)])
        pltpu.sync_copy(z1_hbm, dbuf_v)
        pltpu.sync_copy(dbuf_v, deg_sh.at[pl.ds(sid * DEG_R, DEG_R)])

    pltpu.sync_copy(edges_hbm.at[0, wid], src_v)
    pltpu.sync_copy(edges_hbm.at[1, wid], dst_v)
    for j in range(8):
        ones_v[pl.ds(16 * j, 16)] = jnp.full((16,), 1.0, dtype=jnp.float32)

    plsc.subcore_barrier()

    def sidx(c):
        return src_v.at[c]

    def didx(c):
        return dst_v.at[c]

    for c0 in range(LA):
        pltpu.async_copy(y_sh.at[sidx(c0)], rows_v.at[c0], gsems[c0])

    def group(g, carry):
        for b in range(NBUF):
            c = NBUF * g + b
            b2 = (b + LA) % NBUF

            pltpu.make_async_copy(y_sh.at[sidx(c)], rows_v.at[b],
                                  gsems[b]).wait()
            pltpu.async_copy(rows_v.at[b], acc_sh.at[didx(c)],
                             ssems[b], add=True)
            pltpu.async_copy(ones_v.at[pl.ds(0, CHUNK)],
                             deg_sh.at[didx(c)], dsems[b], add=True)

            @pl.when(c >= LA)
            def _():
                pltpu.make_async_copy(rows_v.at[b2],
                                      acc_sh.at[didx(c - LA)],
                                      ssems[b2]).wait()
                pltpu.make_async_copy(ones_v.at[pl.ds(0, CHUNK)],
                                      deg_sh.at[didx(c - LA)],
                                      dsems[b2]).wait()

            @pl.when(c < NCHUNK - LA)
            def _():
                pltpu.async_copy(y_sh.at[sidx(c + LA)], rows_v.at[b2],
                                 gsems[b2])
        return carry

    lax.fori_loop(0, NCHUNK // NBUF, group, 0)

    for j in range(LA):
        c = NCHUNK - LA + j
        b = c % NBUF
        pltpu.make_async_copy(rows_v.at[b], acc_sh.at[didx(c)],
                              ssems[b]).wait()
        pltpu.make_async_copy(ones_v.at[pl.ds(0, CHUNK)],
                              deg_sh.at[didx(c)], dsems[b]).wait()

    plsc.subcore_barrier()

    @pl.when(sid < DEG_T)
    def _():
        pltpu.sync_copy(acc_sh.at[pl.ds(sid * DEG_R, DEG_R)],
                        acc_out.at[pl.ds(sid * DEG_R, DEG_R),
                                   pl.ds(cid * H, H)])
        pltpu.sync_copy(deg_sh.at[pl.ds(sid * DEG_R, DEG_R)], dbuf_v)

        @pl.when(cid == 0)
        def _():
            pltpu.sync_copy(dbuf_v, deg0_out.at[pl.ds(sid * DEG_R, DEG_R)])

        @pl.when(cid == 1)
        def _():
            pltpu.sync_copy(dbuf_v, deg1_out.at[pl.ds(sid * DEG_R, DEG_R)])


def _sc_aggregate(y2, edges, z2, z1):
    mesh = plsc.VectorSubcoreMesh(core_axis_name="c", subcore_axis_name="s")
    f = pl.kernel(
        _sc_body,
        out_type=(
            jax.ShapeDtypeStruct((N, H2), jnp.float32),
            jax.ShapeDtypeStruct((N,), jnp.float32),
            jax.ShapeDtypeStruct((N,), jnp.float32),
        ),
        mesh=mesh,
        compiler_params=pltpu.CompilerParams(use_tc_tiling_on_sc=False),
        scratch_types=[
            pltpu.VMEM((NCHUNK, CHUNK), jnp.int32),
            pltpu.VMEM((NCHUNK, CHUNK), jnp.int32),
            pltpu.VMEM((NBUF, CHUNK, H), jnp.float32),
            pltpu.VMEM((128,), jnp.float32),
            pltpu.VMEM((DEG_R,), jnp.float32),
            pltpu.VMEM_SHARED((N, H), jnp.float32),
            pltpu.VMEM_SHARED((N,), jnp.float32),
            pltpu.VMEM_SHARED((N, H), jnp.float32),
        ] + [pltpu.SemaphoreType.DMA] * (3 * NBUF),
    )
    return f(y2, edges, z2, z1)


def _head_body(acc_ref, yr_ref, deg0_ref, deg1_ref, wh_ref, bh_ref, out_ref):
    a = acc_ref[:, :H] + acc_ref[:, H:]
    row = pl.program_id(0)
    dsum = deg0_ref[row] + deg1_ref[row]
    scale = 1.0 / jnp.maximum(dsum, 1.0)
    z = jnp.maximum(a * scale[:, None] + yr_ref[:, H:], 0.0)
    out_ref[...] = (
        jnp.dot(z, wh_ref[...], preferred_element_type=jnp.float32)
        + bh_ref[...][None, :]
    )


def _head(acc, yr, deg0, deg1, W_head, b_head):
    BN = 2000
    return pl.pallas_call(
        _head_body,
        grid=(N // BN,),
        in_specs=[
            pl.BlockSpec((BN, H2), lambda i: (i, 0)),
            pl.BlockSpec((BN, H2), lambda i: (i, 0)),
            pl.BlockSpec((N // BN, BN), lambda i: (0, 0)),
            pl.BlockSpec((N // BN, BN), lambda i: (0, 0)),
            pl.BlockSpec((H, O), lambda i: (0, 0)),
            pl.BlockSpec((O,), lambda i: (0,)),
        ],
        out_specs=pl.BlockSpec((BN, O), lambda i: (i, 0)),
        out_shape=jax.ShapeDtypeStruct((N, O), jnp.float32),
    )(acc, yr, deg0, deg1, W_head, b_head)


def kernel(x, edge_index, W_l, b_l, W_r, W_head, b_head):
    edges = edge_index.reshape(2, NW, NCHUNK, CHUNK)
    yr = _project(x, W_l, W_r, b_l)
    z2 = jnp.zeros((DEG_R, H), jnp.float32)
    z1 = jnp.zeros((DEG_R,), jnp.float32)
    acc, deg0, deg1 = _sc_aggregate(yr, edges, z2, z1)
    out = _head(acc, yr, deg0.reshape(5, 2000), deg1.reshape(5, 2000),
                W_head, b_head)
    return out


# speedup vs baseline: 1.5210x; 1.5210x over previous
import functools

import jax
import jax.numpy as jnp
from jax import lax
from jax.experimental import pallas as pl
from jax.experimental.pallas import tpu as pltpu
from jax.experimental.pallas import tpu_sc as plsc

N, E, D, H, O = 10000, 320000, 128, 64, 2
H2 = 2 * H
NC, NS = 2, 16
NW = NC * NS
EPW = E // NW
CHUNK = 125
NCHUNK = EPW // CHUNK
DEG_T = 10
DEG_R = N // DEG_T


def _project_body(x_ref, wl_ref, wr_ref, bl_ref, yr_ref):
    xb = x_ref[...]
    y = jnp.dot(xb, wl_ref[...], preferred_element_type=jnp.float32)
    r = (
        jnp.dot(xb, wr_ref[...], preferred_element_type=jnp.float32)
        + bl_ref[...][None, :]
    )
    yr_ref[...] = jnp.concatenate([y, r], axis=1)


def _project(x, W_l, W_r, b_l):
    BN = 2000
    return pl.pallas_call(
        _project_body,
        grid=(N // BN,),
        in_specs=[
            pl.BlockSpec((BN, D), lambda i: (i, 0)),
            pl.BlockSpec((D, H), lambda i: (0, 0)),
            pl.BlockSpec((D, H), lambda i: (0, 0)),
            pl.BlockSpec((H,), lambda i: (0,)),
        ],
        out_specs=pl.BlockSpec((BN, H2), lambda i: (i, 0)),
        out_shape=jax.ShapeDtypeStruct((N, H2), jnp.float32),
    )(x, W_l, W_r, b_l)


NBUF = 8
LA = 6
DR = NBUF - LA


def _sc_body(y_hbm, edges_hbm, z2_hbm, z1_hbm,
             acc_out, deg0_out, deg1_out,
             src_v, dst_v, rows_v, ones_v, dbuf_v, acc_sh, deg_sh,
             *sems):
    gsems = sems[:NBUF]
    ssems = sems[NBUF:2 * NBUF]
    dsems = sems[2 * NBUF:]
    cid = lax.axis_index("c")
    sid = lax.axis_index("s")
    wid = sid * NC + cid

    @pl.when(sid < DEG_T)
    def _():
        pltpu.sync_copy(z2_hbm, acc_sh.at[pl.ds(sid * DEG_R, DEG_R)])
        pltpu.sync_copy(z1_hbm, dbuf_v)
        pltpu.sync_copy(dbuf_v, deg_sh.at[pl.ds(sid * DEG_R, DEG_R)])

    pltpu.sync_copy(edges_hbm.at[0, wid], src_v)
    pltpu.sync_copy(edges_hbm.at[1, wid], dst_v)
    for j in range(8):
        ones_v[pl.ds(16 * j, 16)] = jnp.full((16,), 1.0, dtype=jnp.float32)

    plsc.subcore_barrier()

    def sidx(c):
        return src_v.at[c]

    def didx(c):
        return dst_v.at[c]

    for c0 in range(LA):
        pltpu.async_copy(y_hbm.at[sidx(c0)], rows_v.at[c0], gsems[c0])

    def group(g, carry):
        for b in range(NBUF):
            c = NBUF * g + b
            b2 = (b + LA) % NBUF

            pltpu.make_async_copy(y_hbm.at[sidx(c)], rows_v.at[b],
                                  gsems[b]).wait()
            pltpu.async_copy(rows_v.at[b], acc_sh.at[didx(c)],
                             ssems[b], add=True)
            pltpu.async_copy(ones_v.at[pl.ds(0, CHUNK)],
                             deg_sh.at[didx(c)], dsems[b], add=True)

            @pl.when(c >= DR)
            def _():
                pltpu.make_async_copy(rows_v.at[b2],
                                      acc_sh.at[didx(c - DR)],
                                      ssems[b2]).wait()
                pltpu.make_async_copy(ones_v.at[pl.ds(0, CHUNK)],
                                      deg_sh.at[didx(c - DR)],
                                      dsems[b2]).wait()

            @pl.when(c < NCHUNK - LA)
            def _():
                pltpu.async_copy(y_hbm.at[sidx(c + LA)], rows_v.at[b2],
                                 gsems[b2])
        return carry

    lax.fori_loop(0, NCHUNK // NBUF, group, 0)

    for j in range(DR):
        c = NCHUNK - DR + j
        b = c % NBUF
        pltpu.make_async_copy(rows_v.at[b], acc_sh.at[didx(c)],
                              ssems[b]).wait()
        pltpu.make_async_copy(ones_v.at[pl.ds(0, CHUNK)],
                              deg_sh.at[didx(c)], dsems[b]).wait()

    plsc.subcore_barrier()

    @pl.when(sid < DEG_T)
    def _():
        pltpu.sync_copy(acc_sh.at[pl.ds(sid * DEG_R, DEG_R)],
                        acc_out.at[pl.ds(sid * DEG_R, DEG_R),
                                   pl.ds(cid * H, H)])
        pltpu.sync_copy(deg_sh.at[pl.ds(sid * DEG_R, DEG_R)], dbuf_v)

        @pl.when(cid == 0)
        def _():
            pltpu.sync_copy(dbuf_v, deg0_out.at[pl.ds(sid * DEG_R, DEG_R)])

        @pl.when(cid == 1)
        def _():
            pltpu.sync_copy(dbuf_v, deg1_out.at[pl.ds(sid * DEG_R, DEG_R)])


def _sc_aggregate(y2, edges, z2, z1):
    mesh = plsc.VectorSubcoreMesh(core_axis_name="c", subcore_axis_name="s")
    f = pl.kernel(
        _sc_body,
        out_type=(
            jax.ShapeDtypeStruct((N, H2), jnp.float32),
            jax.ShapeDtypeStruct((N,), jnp.float32),
            jax.ShapeDtypeStruct((N,), jnp.float32),
        ),
        mesh=mesh,
        compiler_params=pltpu.CompilerParams(use_tc_tiling_on_sc=False),
        scratch_types=[
            pltpu.VMEM((NCHUNK, CHUNK), jnp.int32),
            pltpu.VMEM((NCHUNK, CHUNK), jnp.int32),
            pltpu.VMEM((NBUF, CHUNK, H), jnp.float32),
            pltpu.VMEM((128,), jnp.float32),
            pltpu.VMEM((DEG_R,), jnp.float32),
            pltpu.VMEM_SHARED((N, H), jnp.float32),
            pltpu.VMEM_SHARED((N,), jnp.float32),
        ] + [pltpu.SemaphoreType.DMA] * (3 * NBUF),
    )
    return f(y2, edges, z2, z1)


def _head_body(acc_ref, yr_ref, deg0_ref, deg1_ref, wh_ref, bh_ref, out_ref):
    a = acc_ref[:, :H] + acc_ref[:, H:]
    row = pl.program_id(0)
    dsum = deg0_ref[row] + deg1_ref[row]
    scale = 1.0 / jnp.maximum(dsum, 1.0)
    z = jnp.maximum(a * scale[:, None] + yr_ref[:, H:], 0.0)
    out_ref[...] = (
        jnp.dot(z, wh_ref[...], preferred_element_type=jnp.float32)
        + bh_ref[...][None, :]
    )


def _head(acc, yr, deg0, deg1, W_head, b_head):
    BN = 2000
    return pl.pallas_call(
        _head_body,
        grid=(N // BN,),
        in_specs=[
            pl.BlockSpec((BN, H2), lambda i: (i, 0)),
            pl.BlockSpec((BN, H2), lambda i: (i, 0)),
            pl.BlockSpec((N // BN, BN), lambda i: (0, 0)),
            pl.BlockSpec((N // BN, BN), lambda i: (0, 0)),
            pl.BlockSpec((H, O), lambda i: (0, 0)),
            pl.BlockSpec((O,), lambda i: (0,)),
        ],
        out_specs=pl.BlockSpec((BN, O), lambda i: (i, 0)),
        out_shape=jax.ShapeDtypeStruct((N, O), jnp.float32),
    )(acc, yr, deg0, deg1, W_head, b_head)


def kernel(x, edge_index, W_l, b_l, W_r, W_head, b_head):
    mult = jnp.array([[2], [1]], dtype=jnp.int32)
    edges = (edge_index * mult).reshape(2, NW, NCHUNK, CHUNK)
    yr = _project(x, W_l, W_r, b_l)
    y2 = yr.reshape(2 * N, H)
    z2 = jnp.zeros((DEG_R, H), jnp.float32)
    z1 = jnp.zeros((DEG_R,), jnp.float32)
    acc, deg0, deg1 = _sc_aggregate(y2, edges, z2, z1)
    out = _head(acc, yr, deg0.reshape(5, 2000), deg1.reshape(5, 2000),
                W_head, b_head)
    return out
